# baseline (speedup 1.0000x reference)
import jax
import jax.numpy as jnp
from jax.experimental import pallas as pl

_EPS = 1e-8
_ROWS = 100000
_D = 768
_BLK = 4096
_GRID = (_ROWS + _BLK - 1) // _BLK


def _body(u_ref, x_ref, o_ref):
    x = x_ref[...]
    u = u_ref[0, :]
    xu = x * u[None, :]
    xx = x * x
    pd = xu[:, 0:128]
    pn = xx[:, 0:128]
    for g in range(1, _D // 128):
        pd = pd + xu[:, g * 128:(g + 1) * 128]
        pn = pn + xx[:, g * 128:(g + 1) * 128]
    ones = jnp.ones((128, 1), jnp.float32)
    dot = jax.lax.dot_general(
        pd, ones, (((1,), (0,)), ((), ())),
        precision=jax.lax.Precision.DEFAULT,
    )
    nrm = jax.lax.dot_general(
        pn, ones, (((1,), (0,)), ((), ())),
        precision=jax.lax.Precision.DEFAULT,
    )
    nu = jnp.sqrt(jnp.sum(u * u))
    denom = jnp.maximum(nu, _EPS) * jnp.maximum(jnp.sqrt(nrm), _EPS)
    o_ref[...] = dot / denom


def kernel(x, user_embed):
    out = pl.pallas_call(
        _body,
        grid=(_GRID,),
        in_specs=[
            pl.BlockSpec((1, _D), lambda i: (0, 0)),
            pl.BlockSpec((_BLK, _D), lambda i: (i, 0)),
        ],
        out_specs=pl.BlockSpec((_BLK, 1), lambda i: (i, 0)),
        out_shape=jax.ShapeDtypeStruct((_GRID * _BLK, 1), jnp.float32),
    )(user_embed, x)
    return out.reshape(-1)[:_ROWS]


# speedup vs baseline: 1.1042x; 1.1042x over previous
import jax
import jax.numpy as jnp
from jax.experimental import pallas as pl

_EPS = 1e-8
_ROWS = 100000
_D = 768
_BLK = 4096
_NBLK = 25
_GRID = 13


def _body(u_ref, xa_ref, xb_ref, o_ref):
    i = pl.program_id(0)
    u = u_ref[0, :]
    nu = jnp.sqrt(jnp.sum(u * u))
    inv_nu = 1.0 / jnp.maximum(nu, _EPS)
    for xref, row in ((xa_ref, 2 * i), (xb_ref, 2 * i + 1)):
        x = xref[...]
        dot = jnp.sum(x * u[None, :], axis=1)
        nrm = jnp.sum(x * x, axis=1)
        sim = dot * inv_nu / jnp.maximum(jnp.sqrt(nrm), _EPS)
        o_ref[pl.ds(row, 1), :] = sim.reshape(1, _BLK)


def kernel(x, user_embed):
    out = pl.pallas_call(
        _body,
        grid=(_GRID,),
        in_specs=[
            pl.BlockSpec((1, _D), lambda i: (0, 0)),
            pl.BlockSpec((_BLK, _D), lambda i: (2 * i, 0)),
            pl.BlockSpec((_BLK, _D), lambda i: (jnp.minimum(2 * i + 1, _NBLK - 1), 0)),
        ],
        out_specs=pl.BlockSpec((2 * _GRID, _BLK), lambda i: (0, 0)),
        out_shape=jax.ShapeDtypeStruct((2 * _GRID, _BLK), jnp.float32),
    )(user_embed, x, x)
    return out.reshape(-1)[:_ROWS]


# speedup vs baseline: 1.2642x; 1.1449x over previous
import jax
import jax.numpy as jnp
from jax.experimental import pallas as pl

_EPS = 1e-8
_ROWS = 100000
_D = 768
_BLK = 4096
_NBLK = 25
_GRID = 13


def _body(u_ref, xa_ref, xb_ref, o_ref):
    i = pl.program_id(0)
    u = u_ref[0, :]
    nu = jnp.sqrt(jnp.sum(u * u))
    inv_nu = 1.0 / jnp.maximum(nu, _EPS)
    for xref, row in ((xa_ref, 2 * i), (xb_ref, 2 * i + 1)):
        x = xref[...]
        xg = x[:, 0:128]
        pd = xg * u[0:128][None, :]
        pn = xg * xg
        for g in range(1, _D // 128):
            xg = x[:, g * 128:(g + 1) * 128]
            pd = pd + xg * u[g * 128:(g + 1) * 128][None, :]
            pn = pn + xg * xg
        dot = jnp.sum(pd.T, axis=0)
        nrm = jnp.sum(pn.T, axis=0)
        sim = dot * inv_nu / jnp.maximum(jnp.sqrt(nrm), _EPS)
        o_ref[pl.ds(row, 1), :] = sim.reshape(1, _BLK)


def kernel(x, user_embed):
    out = pl.pallas_call(
        _body,
        grid=(_GRID,),
        in_specs=[
            pl.BlockSpec((1, _D), lambda i: (0, 0)),
            pl.BlockSpec((_BLK, _D), lambda i: (2 * i, 0)),
            pl.BlockSpec((_BLK, _D), lambda i: (jnp.minimum(2 * i + 1, _NBLK - 1), 0)),
        ],
        out_specs=pl.BlockSpec((2 * _GRID, _BLK), lambda i: (0, 0)),
        out_shape=jax.ShapeDtypeStruct((2 * _GRID, _BLK), jnp.float32),
    )(user_embed, x, x)
    return out.reshape(-1)[:_ROWS]
